# Initial kernel scaffold; baseline (speedup 1.0000x reference)
import jax
import jax.numpy as jnp
from jax.experimental import pallas as pl


def kernel(pixel_values, conv_w, conv_b, fc1_w, fc1_b, fc2_w, fc2_b, fc3_w, fc3_b, Wc, Wb):
    raise NotImplementedError("write your pallas kernel here")


# speedup vs baseline: 1.0059x; 1.0059x over previous
import jax
import jax.numpy as jnp
from jax.experimental import pallas as pl
from jax.experimental.pallas import tpu as pltpu

_NE, _NQ, _NC = 3, 100, 92
_B, _H, _W = 16, 512, 512


def _router_body(rf_ref, w1_ref, b1_ref, w2_ref, b2_ref, w3_ref, b3_ref,
                 probs_ref, choice_ref):
    h = jnp.maximum(jnp.dot(rf_ref[...], w1_ref[...],
                            preferred_element_type=jnp.float32) + b1_ref[...], 0.0)
    h = jnp.maximum(jnp.dot(h, w2_ref[...],
                            preferred_element_type=jnp.float32) + b2_ref[...], 0.0)
    logits = jnp.dot(h, w3_ref[...], preferred_element_type=jnp.float32) + b3_ref[...]
    m = jnp.max(logits, axis=1, keepdims=True)
    e = jnp.exp(logits - m)
    probs_ref[...] = e / jnp.sum(e, axis=1, keepdims=True)
    l0, l1, l2 = logits[:, 0], logits[:, 1], logits[:, 2]
    c01 = l0 >= l1
    m01 = jnp.where(c01, l0, l1)
    i01 = jnp.where(c01, 0, 1)
    choice_ref[...] = jnp.where(m01 >= l2, i01, 2).astype(jnp.int32)


def kernel(pixel_values, conv_w, conv_b, fc1_w, fc1_b, fc2_w, fc2_b, fc3_w, fc3_b, Wc, Wb):
    Bb = pixel_values.shape[0]
    h = jax.lax.conv_general_dilated(pixel_values, conv_w, window_strides=(4, 4),
                                     padding=((3, 3), (3, 3)),
                                     dimension_numbers=("NCHW", "OIHW", "NCHW"))
    h = h + conv_b[None, :, None, None]
    h = jax.nn.relu(h)
    _, Cc, Hh, Ww = h.shape
    h = h.reshape(Bb, Cc, 4, Hh // 4, 4, Ww // 4).mean(axis=(3, 5))
    rf = h.reshape(Bb, -1)

    routing_probs, expert_choices = pl.pallas_call(
        _router_body,
        out_shape=(jax.ShapeDtypeStruct((_B, _NE), jnp.float32),
                   jax.ShapeDtypeStruct((_B,), jnp.int32)),
    )(rf, fc1_w, fc1_b.reshape(1, 128), fc2_w, fc2_b.reshape(1, 32),
      fc3_w, fc3_b.reshape(1, _NE))

    feat = pixel_values.reshape(Bb, 3, 16, _H // 16, 16, _W // 16).mean(axis=(3, 5)).reshape(Bb, -1)
    onehot = jax.nn.one_hot(expert_choices, _NE, axis=0, dtype=jnp.float32)
    fm = onehot[:, :, None] * feat[None]
    batch_logits = jnp.einsum("ebd,edq->bq", fm, Wc).reshape(Bb, _NQ, _NC)
    batch_boxes = jax.nn.sigmoid(jnp.einsum("ebd,edq->bq", fm, Wb)).reshape(Bb, _NQ, 4)
    return (batch_logits, batch_boxes, routing_probs, expert_choices)


# speedup vs baseline: 3.3130x; 3.2935x over previous
import functools
import jax
import jax.numpy as jnp
from jax.experimental import pallas as pl
from jax.experimental.pallas import tpu as pltpu
from jax.experimental.pallas import tpu_sc as plsc

_NE, _NQ, _NC = 3, 100, 92
_B, _H, _W = 16, 512, 512
_QT = 1152
_NT = 8


def _conv_body(x_ref, wg_ref, cb_ref, pmp_ref, pw_ref, rf_ref, feat_ref, s_ref):
    planes = []
    facc = [None, None, None]
    for e in range(4):
        for c in range(3):
            for f in range(4):
                p = x_ref[0, c, f, pl.Slice(e, 128, 4), :]
                planes.append(p[None])
                facc[c] = p if facc[c] is None else facc[c] + p
    sp = jnp.concatenate(planes, axis=0)
    s_ref[...] = sp.reshape(48, 128 * 128)
    s = s_ref[...]

    h = cb_ref[...] * jnp.ones((32, 16384), jnp.float32)
    lane = jax.lax.broadcasted_iota(jnp.int32, (1, 16384), 1)
    edge = ((lane & 127) != 0).astype(jnp.float32)
    gi = 0
    for dy in (-1, 0):
        for dx in (-1, 0):
            hg = jax.lax.dot_general(wg_ref[gi], s, (((1,), (0,)), ((), ())),
                                     preferred_element_type=jnp.float32)
            sft = -(128 * dy + dx)
            if sft:
                hg = jnp.pad(hg, ((0, 0), (sft, 0)))[:, :16384]
            if dx == -1:
                hg = hg * edge
            h = h + hg
            gi += 1
    h = jnp.maximum(h, 0.0)
    pooled = jax.lax.dot_general(h, pmp_ref[...], (((1,), (0,)), ((), ())),
                                 preferred_element_type=jnp.float32)
    rf_ref[...] = pooled[None]

    frs = []
    for c in range(3):
        frs.append(facc[c].reshape(16, 8, 128).sum(axis=1))
    fr = jnp.concatenate(frs, axis=0)
    fq = jax.lax.dot_general(fr, pw_ref[...], (((1,), (0,)), ((), ())),
                             preferred_element_type=jnp.float32)
    feat_ref[...] = fq[None]


def _router_body(rf_ref, w1_ref, b1_ref, w2_ref, b2_ref, w3_ref, b3_ref,
                 probs_ref, rlt_ref):
    h = jnp.maximum(jnp.dot(rf_ref[...], w1_ref[...],
                            preferred_element_type=jnp.float32) + b1_ref[...], 0.0)
    h = jnp.maximum(jnp.dot(h, w2_ref[...],
                            preferred_element_type=jnp.float32) + b2_ref[...], 0.0)
    logits = jnp.dot(h, w3_ref[...], preferred_element_type=jnp.float32) + b3_ref[...]
    m = jnp.max(logits, axis=1, keepdims=True)
    ex = jnp.exp(logits - m)
    probs_ref[...] = ex / jnp.sum(ex, axis=1, keepdims=True)
    rlt_ref[...] = logits.T


def _sc_route(rlt_hbm, choice_hbm, oh_hbm, l_v, c_v, oh_v):
    wid = jax.lax.axis_index("s") * 2 + jax.lax.axis_index("c")

    @pl.when(wid == 0)
    def _():
        pltpu.sync_copy(rlt_hbm, l_v)
        l0 = l_v[0, :]
        l1 = l_v[1, :]
        l2 = l_v[2, :]
        z = jnp.zeros((16,), jnp.int32)
        c01 = l0 >= l1
        m01 = jnp.where(c01, l0, l1)
        i01 = jnp.where(c01, z, z + 1)
        choice = jnp.where(m01 >= l2, i01, z + 2)
        c_v[...] = choice
        one = jnp.ones((16,), jnp.float32)
        zf = jnp.zeros((16,), jnp.float32)
        oh_v[0, :] = jnp.where(choice == 0, one, zf)
        oh_v[1, :] = jnp.where(choice == 1, one, zf)
        oh_v[2, :] = jnp.where(choice == 2, one, zf)
        pltpu.sync_copy(c_v, choice_hbm)
        pltpu.sync_copy(oh_v, oh_hbm)


def _expert_body(feat_ref, oh_ref, wc_ref, wb_ref, out_ref, bx_ref, fm_ref):
    i = pl.program_id(0)

    @pl.when(i == 0)
    def _masks():
        feat = feat_ref[...]
        ri = jax.lax.broadcasted_iota(jnp.int32, (16, 16), 0)
        ci = jax.lax.broadcasted_iota(jnp.int32, (16, 16), 1)
        eye = (ri == ci).astype(jnp.float32)
        for ee in range(3):
            dm = eye * oh_ref[ee:ee + 1, :]
            fm_ref[ee] = jnp.dot(dm, feat, preferred_element_type=jnp.float32)

    acc = jnp.zeros((16, _QT), jnp.float32)
    for ee in range(3):
        acc += jnp.dot(fm_ref[ee], wc_ref[ee], preferred_element_type=jnp.float32)
    out_ref[...] = acc

    @pl.when(i == _NT - 1)
    def _boxes():
        bacc = jnp.zeros((16, 400), jnp.float32)
        for ee in range(3):
            bacc += jnp.dot(fm_ref[ee], wb_ref[ee], preferred_element_type=jnp.float32)
        bx_ref[...] = jax.nn.sigmoid(bacc)


def _pool_matrix():
    r = jax.lax.broadcasted_iota(jnp.int32, (16384, 16), 0)
    cidx = jax.lax.broadcasted_iota(jnp.int32, (16384, 16), 1)
    blk = ((r // 128) // 32) * 4 + ((r % 128) // 32)
    return jnp.where(blk == cidx, 1.0 / 1024.0, 0.0).astype(jnp.float32)


def kernel(pixel_values, conv_w, conv_b, fc1_w, fc1_b, fc2_w, fc2_b, fc3_w, fc3_b, Wc, Wb):
    xph = pixel_values.reshape(_B, 3, 512, 128, 4).transpose(0, 1, 4, 2, 3)
    cwp = jnp.pad(conv_w, ((0, 0), (0, 0), (1, 0), (1, 0)))
    wgs = []
    for dy in (-1, 0):
        for dx in (-1, 0):
            w = cwp[:, :, 4 * dy + 4:4 * dy + 8, 4 * dx + 4:4 * dx + 8]
            wgs.append(w.transpose(0, 2, 1, 3).reshape(32, 48))
    wg = jnp.stack(wgs)
    pmp = _pool_matrix()
    pw = jnp.where(
        (jax.lax.broadcasted_iota(jnp.int32, (128, 16), 0) // 8)
        == jax.lax.broadcasted_iota(jnp.int32, (128, 16), 1),
        1.0 / 1024.0, 0.0).astype(jnp.float32)

    rf3, feat3 = pl.pallas_call(
        _conv_body,
        grid=(_B,),
        in_specs=[
            pl.BlockSpec((1, 3, 4, 512, 128), lambda b: (b, 0, 0, 0, 0)),
            pl.BlockSpec((4, 32, 48), lambda b: (0, 0, 0)),
            pl.BlockSpec((32, 1), lambda b: (0, 0)),
            pl.BlockSpec((16384, 16), lambda b: (0, 0)),
            pl.BlockSpec((128, 16), lambda b: (0, 0)),
        ],
        out_specs=[
            pl.BlockSpec((1, 32, 16), lambda b: (b, 0, 0)),
            pl.BlockSpec((1, 48, 16), lambda b: (b, 0, 0)),
        ],
        out_shape=[
            jax.ShapeDtypeStruct((_B, 32, 16), jnp.float32),
            jax.ShapeDtypeStruct((_B, 48, 16), jnp.float32),
        ],
        scratch_shapes=[
            pltpu.VMEM((48, 16384), jnp.float32),
        ],
    )(xph, wg, conv_b.reshape(32, 1), pmp, pw)
    rf = rf3.reshape(_B, 512)
    feat = feat3.reshape(_B, 768)

    routing_probs, rlt = pl.pallas_call(
        _router_body,
        out_shape=(jax.ShapeDtypeStruct((_B, _NE), jnp.float32),
                   jax.ShapeDtypeStruct((_NE, _B), jnp.float32)),
    )(rf, fc1_w, fc1_b.reshape(1, 128), fc2_w, fc2_b.reshape(1, 32),
      fc3_w, fc3_b.reshape(1, _NE))

    sc_route = functools.partial(
        pl.kernel,
        out_type=[
            jax.ShapeDtypeStruct((_B,), jnp.int32),
            jax.ShapeDtypeStruct((_NE, _B), jnp.float32),
        ],
        mesh=plsc.VectorSubcoreMesh(core_axis_name="c", subcore_axis_name="s"),
        scratch_types=[
            pltpu.VMEM((_NE, _B), jnp.float32),
            pltpu.VMEM((_B,), jnp.int32),
            pltpu.VMEM((_NE, _B), jnp.float32),
        ],
    )(_sc_route)
    expert_choices, oh = sc_route(rlt)

    batch_logits, batch_boxes = pl.pallas_call(
        _expert_body,
        grid=(_NT,),
        in_specs=[
            pl.BlockSpec((_B, 768), lambda i: (0, 0)),
            pl.BlockSpec((_NE, _B), lambda i: (0, 0)),
            pl.BlockSpec((_NE, 768, _QT), lambda i: (0, 0, i)),
            pl.BlockSpec((_NE, 768, 400), lambda i: (0, 0, 0)),
        ],
        out_specs=[
            pl.BlockSpec((_B, _QT), lambda i: (0, i)),
            pl.BlockSpec((_B, 400), lambda i: (0, 0)),
        ],
        out_shape=[
            jax.ShapeDtypeStruct((_B, 9200), jnp.float32),
            jax.ShapeDtypeStruct((_B, 400), jnp.float32),
        ],
        scratch_shapes=[pltpu.VMEM((_NE, _B, 768), jnp.float32)],
    )(feat, oh, Wc, Wb)

    return (batch_logits.reshape(_B, _NQ, _NC),
            batch_boxes.reshape(_B, _NQ, 4),
            routing_probs,
            expert_choices)
